# initial kernel scaffold (unmeasured)
import jax
import jax.numpy as jnp
from jax import lax
from jax.experimental import pallas as pl
from jax.experimental.pallas import tpu as pltpu


def kernel(
    x,
):
    def body(*refs):
        pass

    out_shape = jax.ShapeDtypeStruct(..., jnp.float32)
    return pl.pallas_call(body, out_shape=out_shape)(...)


# baseline (device time: 592879 ns/iter reference)
import jax
import jax.numpy as jnp
from jax import lax
from jax.experimental import pallas as pl
from jax.experimental.pallas import tpu as pltpu

N_DEV = 4


def kernel(x):
    _, m, n_tot = x.shape
    chunk = n_tot // N_DEV
    half_m = m // 2

    def body(x_hbm, out_hbm, comm_ref, stage_ref, copy_sem, send_sems, recv_sems):
        my_x = lax.axis_index("x")
        my_y = lax.axis_index("y")
        my_z = lax.axis_index("z")
        right = (my_y + 1) % N_DEV
        left = (my_y + N_DEV - 1) % N_DEV

        barrier = pltpu.get_barrier_semaphore()
        for nbr in (left, right):
            pl.semaphore_signal(
                barrier, inc=1,
                device_id=(my_x, nbr, my_z),
                device_id_type=pl.DeviceIdType.MESH,
            )
        pl.semaphore_wait(barrier, 2)

        def accumulate_chunk(c, slot):
            for r in range(2):
                rows = pl.ds(r * half_m, half_m)
                cp = pltpu.make_async_copy(
                    x_hbm.at[0, rows, pl.ds(c * chunk, chunk)],
                    stage_ref,
                    copy_sem,
                )
                cp.start()
                cp.wait()
                comm_ref[slot, rows, :] += stage_ref[...]

        c0 = (my_y + N_DEV - 1) % N_DEV
        rdma = pltpu.make_async_remote_copy(
            src_ref=x_hbm.at[0, :, pl.ds(c0 * chunk, chunk)],
            dst_ref=comm_ref.at[0],
            send_sem=send_sems.at[0],
            recv_sem=recv_sems.at[0],
            device_id=(my_x, right, my_z),
            device_id_type=pl.DeviceIdType.MESH,
        )
        rdma.start()
        rdma.wait()

        for s in range(1, N_DEV - 1):
            c = (my_y + N_DEV - 1 - s) % N_DEV
            accumulate_chunk(c, s - 1)
            rdma = pltpu.make_async_remote_copy(
                src_ref=comm_ref.at[s - 1],
                dst_ref=comm_ref.at[s],
                send_sem=send_sems.at[s],
                recv_sem=recv_sems.at[s],
                device_id=(my_x, right, my_z),
                device_id_type=pl.DeviceIdType.MESH,
            )
            rdma.start()
            rdma.wait()

        accumulate_chunk(my_y, N_DEV - 2)
        cp = pltpu.make_async_copy(comm_ref.at[N_DEV - 2], out_hbm, copy_sem)
        cp.start()
        cp.wait()

    return pl.pallas_call(
        body,
        out_shape=jax.ShapeDtypeStruct((m, chunk), jnp.float32),
        in_specs=[pl.BlockSpec(memory_space=pl.ANY)],
        out_specs=pl.BlockSpec(memory_space=pl.ANY),
        scratch_shapes=[
            pltpu.VMEM((N_DEV - 1, m, chunk), jnp.float32),
            pltpu.VMEM((half_m, chunk), jnp.float32),
            pltpu.SemaphoreType.DMA,
            pltpu.SemaphoreType.DMA((N_DEV - 1,)),
            pltpu.SemaphoreType.DMA((N_DEV - 1,)),
        ],
        compiler_params=pltpu.CompilerParams(
            collective_id=0,
            vmem_limit_bytes=63 * 1024 * 1024,
        ),
    )(x)


# device time: 262582 ns/iter; 2.2579x vs baseline; 2.2579x over previous
import jax
import jax.numpy as jnp
from jax import lax
from jax.experimental import pallas as pl
from jax.experimental.pallas import tpu as pltpu

N_Y = 4
N_Z = 4
N_X = 2


def kernel(x):
    _, m, n_tot = x.shape
    chunk = n_tot // N_Y
    strip = m // (N_X * N_Z)
    half = m // N_X

    def body(
        x_hbm, out_ref, comm_ref, stage_ref,
        copy_sem, p1_send, p1_recv,
        ze_send, ze_recv, zw_send, zw_recv,
        x_send, x_recv,
    ):
        my_x = lax.axis_index("x")
        my_y = lax.axis_index("y")
        my_z = lax.axis_index("z")
        right = (my_y + 1) % N_Y
        left = (my_y + N_Y - 1) % N_Y

        row0 = (N_Z * my_x + my_z) * strip

        barrier = pltpu.get_barrier_semaphore()
        pl.semaphore_signal(barrier, inc=1, device_id=(my_x, left, my_z),
                            device_id_type=pl.DeviceIdType.MESH)
        pl.semaphore_signal(barrier, inc=1, device_id=(my_x, right, my_z),
                            device_id_type=pl.DeviceIdType.MESH)
        pl.semaphore_signal(barrier, inc=1, device_id=(1 - my_x, my_y, my_z),
                            device_id_type=pl.DeviceIdType.MESH)

        @pl.when(my_z > 0)
        def _():
            pl.semaphore_signal(barrier, inc=1, device_id=(my_x, my_y, my_z - 1),
                                device_id_type=pl.DeviceIdType.MESH)

        @pl.when(my_z < N_Z - 1)
        def _():
            pl.semaphore_signal(barrier, inc=1, device_id=(my_x, my_y, my_z + 1),
                                device_id_type=pl.DeviceIdType.MESH)

        n_nbrs = 3 + (my_z > 0).astype(jnp.int32) + (my_z < N_Z - 1).astype(jnp.int32)
        pl.semaphore_wait(barrier, n_nbrs)

        def stage_start(c):
            cp = pltpu.make_async_copy(
                x_hbm.at[0, pl.ds(row0, strip), pl.ds(c * chunk, chunk)],
                stage_ref, copy_sem,
            )
            cp.start()
            return cp

        def hop_rdma(s, src_ref):
            return pltpu.make_async_remote_copy(
                src_ref=src_ref,
                dst_ref=comm_ref.at[s],
                send_sem=p1_send.at[s],
                recv_sem=p1_recv.at[s],
                device_id=(my_x, right, my_z),
                device_id_type=pl.DeviceIdType.MESH,
            )

        c0 = (my_y + N_Y - 1) % N_Y
        rdma = hop_rdma(
            0, x_hbm.at[0, pl.ds(row0, strip), pl.ds(c0 * chunk, chunk)]
        )
        rdma.start()
        cp = stage_start((my_y + N_Y - 2) % N_Y)
        rdma.wait()

        for s in (1, 2):
            cp.wait()
            comm_ref[s - 1] += stage_ref[...]
            rdma = hop_rdma(s, comm_ref.at[s - 1])
            rdma.start()
            cp = stage_start((my_y + N_Y - 2 - s) % N_Y)
            rdma.wait()

        cp.wait()
        out_ref[pl.ds(row0, strip), :] = comm_ref[2] + stage_ref[...]

        def strip_rows(o):
            return pl.ds((N_Z * my_x + o) * strip, strip)

        def z_rdma(origin, dz, send_s, recv_s):
            return pltpu.make_async_remote_copy(
                src_ref=out_ref.at[strip_rows(origin), :],
                dst_ref=out_ref.at[strip_rows(origin), :],
                send_sem=send_s,
                recv_sem=recv_s,
                device_id=(my_x, my_y, my_z + dz),
                device_id_type=pl.DeviceIdType.MESH,
            )

        for t in range(N_Z - 1):
            e_send = (my_z < N_Z - 1) & (my_z >= t)
            w_send = (my_z > 0) & (my_z + t <= N_Z - 1)

            @pl.when(e_send)
            def _():
                z_rdma(my_z - t, 1, ze_send.at[t], ze_recv.at[t]).start()

            @pl.when(w_send)
            def _():
                z_rdma(my_z + t, -1, zw_send.at[t], zw_recv.at[t]).start()

            @pl.when(my_z >= t + 1)
            def _():
                z_rdma(my_z - 1 - t, -1, ze_send.at[t], ze_recv.at[t]).wait_recv()

            @pl.when(my_z <= N_Z - 2 - t)
            def _():
                z_rdma(my_z + 1 + t, 1, zw_send.at[t], zw_recv.at[t]).wait_recv()

            @pl.when(e_send)
            def _():
                z_rdma(my_z - t, 1, ze_send.at[t], ze_recv.at[t]).wait_send()

            @pl.when(w_send)
            def _():
                z_rdma(my_z + t, -1, zw_send.at[t], zw_recv.at[t]).wait_send()

        xr = pltpu.make_async_remote_copy(
            src_ref=out_ref.at[pl.ds(half * my_x, half), :],
            dst_ref=out_ref.at[pl.ds(half * my_x, half), :],
            send_sem=x_send,
            recv_sem=x_recv,
            device_id=(1 - my_x, my_y, my_z),
            device_id_type=pl.DeviceIdType.MESH,
        )
        xr.start()
        xr.wait()

    return pl.pallas_call(
        body,
        out_shape=jax.ShapeDtypeStruct((m, chunk), jnp.float32),
        in_specs=[pl.BlockSpec(memory_space=pl.ANY)],
        out_specs=pl.BlockSpec(memory_space=pltpu.MemorySpace.VMEM),
        scratch_shapes=[
            pltpu.VMEM((N_Y - 1, strip, chunk), jnp.float32),
            pltpu.VMEM((strip, chunk), jnp.float32),
            pltpu.SemaphoreType.DMA,
            pltpu.SemaphoreType.DMA((N_Y - 1,)),
            pltpu.SemaphoreType.DMA((N_Y - 1,)),
            pltpu.SemaphoreType.DMA((N_Z - 1,)),
            pltpu.SemaphoreType.DMA((N_Z - 1,)),
            pltpu.SemaphoreType.DMA((N_Z - 1,)),
            pltpu.SemaphoreType.DMA((N_Z - 1,)),
            pltpu.SemaphoreType.DMA,
            pltpu.SemaphoreType.DMA,
        ],
        compiler_params=pltpu.CompilerParams(
            collective_id=0,
            vmem_limit_bytes=63 * 1024 * 1024,
        ),
    )(x)


# device time: 195201 ns/iter; 3.0373x vs baseline; 1.3452x over previous
import jax
import jax.numpy as jnp
from jax import lax
from jax.experimental import pallas as pl
from jax.experimental.pallas import tpu as pltpu

N_Y = 4
N_Z = 4
N_X = 2


def kernel(x):
    _, m, n_tot = x.shape
    chunk = n_tot // N_Y
    strip = m // (N_X * N_Z)
    half = m // N_X

    def body(
        x_hbm, out_ref, comm_ref, stage_ref,
        copy_sem, p1_send, p1_recv,
        ze_send, ze_recv, zw_send, zw_recv,
        x_send, x_recv,
    ):
        my_x = lax.axis_index("x")
        my_y = lax.axis_index("y")
        my_z = lax.axis_index("z")
        right = (my_y + 1) % N_Y
        left = (my_y + N_Y - 1) % N_Y

        row0 = (N_Z * my_x + my_z) * strip

        barrier = pltpu.get_barrier_semaphore()
        pl.semaphore_signal(barrier, inc=1, device_id=(my_x, left, my_z),
                            device_id_type=pl.DeviceIdType.MESH)
        pl.semaphore_signal(barrier, inc=1, device_id=(my_x, right, my_z),
                            device_id_type=pl.DeviceIdType.MESH)
        pl.semaphore_signal(barrier, inc=1, device_id=(1 - my_x, my_y, my_z),
                            device_id_type=pl.DeviceIdType.MESH)

        @pl.when(my_z > 0)
        def _():
            pl.semaphore_signal(barrier, inc=1, device_id=(my_x, my_y, my_z - 1),
                                device_id_type=pl.DeviceIdType.MESH)

        @pl.when(my_z < N_Z - 1)
        def _():
            pl.semaphore_signal(barrier, inc=1, device_id=(my_x, my_y, my_z + 1),
                                device_id_type=pl.DeviceIdType.MESH)

        n_nbrs = 3 + (my_z > 0).astype(jnp.int32) + (my_z < N_Z - 1).astype(jnp.int32)
        pl.semaphore_wait(barrier, n_nbrs)

        def stage_start(c):
            cp = pltpu.make_async_copy(
                x_hbm.at[0, pl.ds(row0, strip), pl.ds(c * chunk, chunk)],
                stage_ref, copy_sem,
            )
            cp.start()
            return cp

        def hop_rdma(s, src_ref):
            return pltpu.make_async_remote_copy(
                src_ref=src_ref,
                dst_ref=comm_ref.at[s],
                send_sem=p1_send.at[s],
                recv_sem=p1_recv.at[s],
                device_id=(my_x, right, my_z),
                device_id_type=pl.DeviceIdType.MESH,
            )

        c0 = (my_y + N_Y - 1) % N_Y
        rdma = hop_rdma(
            0, x_hbm.at[0, pl.ds(row0, strip), pl.ds(c0 * chunk, chunk)]
        )
        rdma.start()
        cp = stage_start((my_y + N_Y - 2) % N_Y)
        rdma.wait()

        for s in (1, 2):
            cp.wait()
            comm_ref[s - 1] += stage_ref[...]
            rdma = hop_rdma(s, comm_ref.at[s - 1])
            rdma.start()
            cp = stage_start((my_y + N_Y - 2 - s) % N_Y)
            rdma.wait()

        cp.wait()
        out_ref[pl.ds(row0, strip), :] = comm_ref[2] + stage_ref[...]

        def strip_rows(o):
            return pl.ds((N_Z * my_x + o) * strip, strip)

        def z_rdma(origin, dz, send_s, recv_s):
            return pltpu.make_async_remote_copy(
                src_ref=out_ref.at[strip_rows(origin), :],
                dst_ref=out_ref.at[strip_rows(origin), :],
                send_sem=send_s,
                recv_sem=recv_s,
                device_id=(my_x, my_y, my_z + dz),
                device_id_type=pl.DeviceIdType.MESH,
            )

        def x_rdma_send(origin):
            return pltpu.make_async_remote_copy(
                src_ref=out_ref.at[strip_rows(origin), :],
                dst_ref=out_ref.at[strip_rows(origin), :],
                send_sem=x_send.at[origin],
                recv_sem=x_recv.at[origin],
                device_id=(1 - my_x, my_y, my_z),
                device_id_type=pl.DeviceIdType.MESH,
            )

        def x_rdma_recv(origin):
            rows = pl.ds((N_Z * (1 - my_x) + origin) * strip, strip)
            return pltpu.make_async_remote_copy(
                src_ref=out_ref.at[rows, :],
                dst_ref=out_ref.at[rows, :],
                send_sem=x_send.at[origin],
                recv_sem=x_recv.at[origin],
                device_id=(1 - my_x, my_y, my_z),
                device_id_type=pl.DeviceIdType.MESH,
            )

        x_rdma_send(my_z).start()

        for t in range(N_Z - 1):
            e_send = (my_z < N_Z - 1) & (my_z >= t)
            w_send = (my_z > 0) & (my_z + t <= N_Z - 1)

            @pl.when(e_send)
            def _():
                z_rdma(my_z - t, 1, ze_send.at[t], ze_recv.at[t]).start()

            @pl.when(w_send)
            def _():
                z_rdma(my_z + t, -1, zw_send.at[t], zw_recv.at[t]).start()

            @pl.when(my_z >= t + 1)
            def _():
                z_rdma(my_z - 1 - t, -1, ze_send.at[t], ze_recv.at[t]).wait_recv()
                x_rdma_send(my_z - 1 - t).start()

            @pl.when(my_z <= N_Z - 2 - t)
            def _():
                z_rdma(my_z + 1 + t, 1, zw_send.at[t], zw_recv.at[t]).wait_recv()
                x_rdma_send(my_z + 1 + t).start()

            @pl.when(e_send)
            def _():
                z_rdma(my_z - t, 1, ze_send.at[t], ze_recv.at[t]).wait_send()

            @pl.when(w_send)
            def _():
                z_rdma(my_z + t, -1, zw_send.at[t], zw_recv.at[t]).wait_send()

        for o in range(N_Z):
            x_rdma_recv(o).wait_recv()
            x_rdma_send(o).wait_send()

    return pl.pallas_call(
        body,
        out_shape=jax.ShapeDtypeStruct((m, chunk), jnp.float32),
        in_specs=[pl.BlockSpec(memory_space=pl.ANY)],
        out_specs=pl.BlockSpec(memory_space=pltpu.MemorySpace.VMEM),
        scratch_shapes=[
            pltpu.VMEM((N_Y - 1, strip, chunk), jnp.float32),
            pltpu.VMEM((strip, chunk), jnp.float32),
            pltpu.SemaphoreType.DMA,
            pltpu.SemaphoreType.DMA((N_Y - 1,)),
            pltpu.SemaphoreType.DMA((N_Y - 1,)),
            pltpu.SemaphoreType.DMA((N_Z - 1,)),
            pltpu.SemaphoreType.DMA((N_Z - 1,)),
            pltpu.SemaphoreType.DMA((N_Z - 1,)),
            pltpu.SemaphoreType.DMA((N_Z - 1,)),
            pltpu.SemaphoreType.DMA((N_Z,)),
            pltpu.SemaphoreType.DMA((N_Z,)),
        ],
        compiler_params=pltpu.CompilerParams(
            collective_id=0,
            vmem_limit_bytes=63 * 1024 * 1024,
        ),
    )(x)


# device time: 161405 ns/iter; 3.6732x vs baseline; 1.2094x over previous
import jax
import jax.numpy as jnp
from jax import lax
from jax.experimental import pallas as pl
from jax.experimental.pallas import tpu as pltpu

N_Y = 4
N_Z = 4
N_X = 2
K = 2


def kernel(x):
    _, m, n_tot = x.shape
    chunk = n_tot // N_Y
    strip = m // (N_X * N_Z)
    sub = strip // K

    def body(
        x_hbm, out_ref, comm_ref, stage_ref,
        copy_sems, p1_send, p1_recv,
        ze_send, ze_recv, zw_send, zw_recv,
        x_send, x_recv,
    ):
        my_x = lax.axis_index("x")
        my_y = lax.axis_index("y")
        my_z = lax.axis_index("z")
        right = (my_y + 1) % N_Y
        left = (my_y + N_Y - 1) % N_Y

        row0 = (N_Z * my_x + my_z) * strip

        barrier = pltpu.get_barrier_semaphore()
        pl.semaphore_signal(barrier, inc=1, device_id=(my_x, left, my_z),
                            device_id_type=pl.DeviceIdType.MESH)
        pl.semaphore_signal(barrier, inc=1, device_id=(my_x, right, my_z),
                            device_id_type=pl.DeviceIdType.MESH)
        pl.semaphore_signal(barrier, inc=1, device_id=(1 - my_x, my_y, my_z),
                            device_id_type=pl.DeviceIdType.MESH)

        @pl.when(my_z > 0)
        def _():
            pl.semaphore_signal(barrier, inc=1, device_id=(my_x, my_y, my_z - 1),
                                device_id_type=pl.DeviceIdType.MESH)

        @pl.when(my_z < N_Z - 1)
        def _():
            pl.semaphore_signal(barrier, inc=1, device_id=(my_x, my_y, my_z + 1),
                                device_id_type=pl.DeviceIdType.MESH)

        n_nbrs = 3 + (my_z > 0).astype(jnp.int32) + (my_z < N_Z - 1).astype(jnp.int32)
        pl.semaphore_wait(barrier, n_nbrs)

        def addend(s):
            return (my_y + N_Y - 1 - s) % N_Y

        def stage_copy(k, c):
            return pltpu.make_async_copy(
                x_hbm.at[0, pl.ds(row0 + k * sub, sub), pl.ds(c * chunk, chunk)],
                stage_ref.at[k],
                copy_sems.at[k],
            )

        def hop_desc(k, s):
            if s == 0:
                src = x_hbm.at[0, pl.ds(row0 + k * sub, sub),
                               pl.ds(addend(0) * chunk, chunk)]
            else:
                src = comm_ref.at[k, s - 1]
            return pltpu.make_async_remote_copy(
                src_ref=src,
                dst_ref=comm_ref.at[k, s],
                send_sem=p1_send.at[k, s],
                recv_sem=p1_recv.at[k, s],
                device_id=(my_x, right, my_z),
                device_id_type=pl.DeviceIdType.MESH,
            )

        def p1_start(k, s):
            if s > 0:
                stage_copy(k, addend(s)).wait()
                comm_ref[k, s - 1] += stage_ref[k]
            hop_desc(k, s).start()
            stage_copy(k, addend(s + 1)).start()

        def p1_wait(k, s):
            hop_desc(k, s).wait()

        def p1_finish(k):
            stage_copy(k, my_y).wait()
            out_ref[pl.ds(row0 + k * sub, sub), :] = comm_ref[k, N_Y - 2] + stage_ref[k]
            x_push(k, my_z).start()

        def piece_rows(o, k, xside=None):
            xi = my_x if xside is None else xside
            return pl.ds((N_Z * xi + o) * strip + k * sub, sub)

        def z_rdma(k, origin, dz, send_s, recv_s):
            return pltpu.make_async_remote_copy(
                src_ref=out_ref.at[piece_rows(origin, k), :],
                dst_ref=out_ref.at[piece_rows(origin, k), :],
                send_sem=send_s,
                recv_sem=recv_s,
                device_id=(my_x, my_y, my_z + dz),
                device_id_type=pl.DeviceIdType.MESH,
            )

        def x_push(k, origin):
            return pltpu.make_async_remote_copy(
                src_ref=out_ref.at[piece_rows(origin, k), :],
                dst_ref=out_ref.at[piece_rows(origin, k), :],
                send_sem=x_send.at[k, origin],
                recv_sem=x_recv.at[k, origin],
                device_id=(1 - my_x, my_y, my_z),
                device_id_type=pl.DeviceIdType.MESH,
            )

        def x_recv_desc(k, origin):
            rows = piece_rows(origin, k, xside=1 - my_x)
            return pltpu.make_async_remote_copy(
                src_ref=out_ref.at[rows, :],
                dst_ref=out_ref.at[rows, :],
                send_sem=x_send.at[k, origin],
                recv_sem=x_recv.at[k, origin],
                device_id=(1 - my_x, my_y, my_z),
                device_id_type=pl.DeviceIdType.MESH,
            )

        def z_sends(k, t):
            @pl.when((my_z < N_Z - 1) & (my_z >= t))
            def _():
                z_rdma(k, my_z - t, 1, ze_send.at[k, t], ze_recv.at[k, t]).start()

            @pl.when((my_z > 0) & (my_z + t <= N_Z - 1))
            def _():
                z_rdma(k, my_z + t, -1, zw_send.at[k, t], zw_recv.at[k, t]).start()

        def z_waits(k, t):
            @pl.when(my_z >= t + 1)
            def _():
                z_rdma(k, my_z - 1 - t, -1, ze_send.at[k, t], ze_recv.at[k, t]).wait_recv()
                x_push(k, my_z - 1 - t).start()

            @pl.when(my_z <= N_Z - 2 - t)
            def _():
                z_rdma(k, my_z + 1 + t, 1, zw_send.at[k, t], zw_recv.at[k, t]).wait_recv()
                x_push(k, my_z + 1 + t).start()

            @pl.when((my_z < N_Z - 1) & (my_z >= t))
            def _():
                z_rdma(k, my_z - t, 1, ze_send.at[k, t], ze_recv.at[k, t]).wait_send()

            @pl.when((my_z > 0) & (my_z + t <= N_Z - 1))
            def _():
                z_rdma(k, my_z + t, -1, zw_send.at[k, t], zw_recv.at[k, t]).wait_send()

        for s in range(N_Y - 1):
            p1_start(0, s)
            p1_wait(0, s)
        p1_finish(0)

        z_sends(0, 0)
        p1_start(1, 0)
        z_waits(0, 0)
        z_sends(0, 1)
        p1_wait(1, 0)
        p1_start(1, 1)
        z_waits(0, 1)
        z_sends(0, 2)
        p1_wait(1, 1)
        p1_start(1, 2)
        z_waits(0, 2)
        p1_wait(1, 2)
        p1_finish(1)

        for t in range(N_Z - 1):
            z_sends(1, t)
            z_waits(1, t)

        for k in range(K):
            for o in range(N_Z):
                x_recv_desc(k, o).wait_recv()
                x_push(k, o).wait_send()

    return pl.pallas_call(
        body,
        out_shape=jax.ShapeDtypeStruct((m, chunk), jnp.float32),
        in_specs=[pl.BlockSpec(memory_space=pl.ANY)],
        out_specs=pl.BlockSpec(memory_space=pltpu.MemorySpace.VMEM),
        scratch_shapes=[
            pltpu.VMEM((K, N_Y - 1, sub, chunk), jnp.float32),
            pltpu.VMEM((K, sub, chunk), jnp.float32),
            pltpu.SemaphoreType.DMA((K,)),
            pltpu.SemaphoreType.DMA((K, N_Y - 1)),
            pltpu.SemaphoreType.DMA((K, N_Y - 1)),
            pltpu.SemaphoreType.DMA((K, N_Z - 1)),
            pltpu.SemaphoreType.DMA((K, N_Z - 1)),
            pltpu.SemaphoreType.DMA((K, N_Z - 1)),
            pltpu.SemaphoreType.DMA((K, N_Z - 1)),
            pltpu.SemaphoreType.DMA((K, N_Z)),
            pltpu.SemaphoreType.DMA((K, N_Z)),
        ],
        compiler_params=pltpu.CompilerParams(
            collective_id=0,
            vmem_limit_bytes=63 * 1024 * 1024,
        ),
    )(x)


# device time: 147391 ns/iter; 4.0225x vs baseline; 1.0951x over previous
import jax
import jax.numpy as jnp
from jax import lax
from jax.experimental import pallas as pl
from jax.experimental.pallas import tpu as pltpu

N_Y = 4
N_Z = 4
N_X = 2
K = 4


def kernel(x):
    _, m, n_tot = x.shape
    chunk = n_tot // N_Y
    strip = m // (N_X * N_Z)
    sub = strip // K

    def body(
        x_hbm, out_ref, comm_ref, stage_ref,
        copy_sems, p1_send, p1_recv,
        ze_send, ze_recv, zw_send, zw_recv,
        x_send, x_recv,
    ):
        my_x = lax.axis_index("x")
        my_y = lax.axis_index("y")
        my_z = lax.axis_index("z")
        right = (my_y + 1) % N_Y
        left = (my_y + N_Y - 1) % N_Y

        row0 = (N_Z * my_x + my_z) * strip

        barrier = pltpu.get_barrier_semaphore()
        pl.semaphore_signal(barrier, inc=1, device_id=(my_x, left, my_z),
                            device_id_type=pl.DeviceIdType.MESH)
        pl.semaphore_signal(barrier, inc=1, device_id=(my_x, right, my_z),
                            device_id_type=pl.DeviceIdType.MESH)
        pl.semaphore_signal(barrier, inc=1, device_id=(1 - my_x, my_y, my_z),
                            device_id_type=pl.DeviceIdType.MESH)

        @pl.when(my_z > 0)
        def _():
            pl.semaphore_signal(barrier, inc=1, device_id=(my_x, my_y, my_z - 1),
                                device_id_type=pl.DeviceIdType.MESH)

        @pl.when(my_z < N_Z - 1)
        def _():
            pl.semaphore_signal(barrier, inc=1, device_id=(my_x, my_y, my_z + 1),
                                device_id_type=pl.DeviceIdType.MESH)

        n_nbrs = 3 + (my_z > 0).astype(jnp.int32) + (my_z < N_Z - 1).astype(jnp.int32)
        pl.semaphore_wait(barrier, n_nbrs)

        def addend(s):
            return (my_y + N_Y - 1 - s) % N_Y

        def stage_copy(k, c):
            return pltpu.make_async_copy(
                x_hbm.at[0, pl.ds(row0 + k * sub, sub), pl.ds(c * chunk, chunk)],
                stage_ref.at[k],
                copy_sems.at[k],
            )

        def hop_desc(k, s):
            if s == 0:
                src = x_hbm.at[0, pl.ds(row0 + k * sub, sub),
                               pl.ds(addend(0) * chunk, chunk)]
            else:
                src = comm_ref.at[k, s - 1]
            return pltpu.make_async_remote_copy(
                src_ref=src,
                dst_ref=comm_ref.at[k, s],
                send_sem=p1_send.at[k, s],
                recv_sem=p1_recv.at[k, s],
                device_id=(my_x, right, my_z),
                device_id_type=pl.DeviceIdType.MESH,
            )

        def p1_start(k, s):
            if s > 0:
                stage_copy(k, addend(s)).wait()
                comm_ref[k, s - 1] += stage_ref[k]
            hop_desc(k, s).start()
            stage_copy(k, addend(s + 1)).start()

        def p1_wait(k, s):
            hop_desc(k, s).wait()

        def p1_finish(k):
            stage_copy(k, my_y).wait()
            out_ref[pl.ds(row0 + k * sub, sub), :] = comm_ref[k, N_Y - 2] + stage_ref[k]
            x_push(k, my_z).start()

        def piece_rows(o, k, xside=None):
            xi = my_x if xside is None else xside
            return pl.ds((N_Z * xi + o) * strip + k * sub, sub)

        def z_rdma(k, origin, dz, send_s, recv_s):
            return pltpu.make_async_remote_copy(
                src_ref=out_ref.at[piece_rows(origin, k), :],
                dst_ref=out_ref.at[piece_rows(origin, k), :],
                send_sem=send_s,
                recv_sem=recv_s,
                device_id=(my_x, my_y, my_z + dz),
                device_id_type=pl.DeviceIdType.MESH,
            )

        def x_push(k, origin):
            return pltpu.make_async_remote_copy(
                src_ref=out_ref.at[piece_rows(origin, k), :],
                dst_ref=out_ref.at[piece_rows(origin, k), :],
                send_sem=x_send.at[k, origin],
                recv_sem=x_recv.at[k, origin],
                device_id=(1 - my_x, my_y, my_z),
                device_id_type=pl.DeviceIdType.MESH,
            )

        def x_recv_desc(k, origin):
            rows = piece_rows(origin, k, xside=1 - my_x)
            return pltpu.make_async_remote_copy(
                src_ref=out_ref.at[rows, :],
                dst_ref=out_ref.at[rows, :],
                send_sem=x_send.at[k, origin],
                recv_sem=x_recv.at[k, origin],
                device_id=(1 - my_x, my_y, my_z),
                device_id_type=pl.DeviceIdType.MESH,
            )

        def z_sends(k, t):
            @pl.when((my_z < N_Z - 1) & (my_z >= t))
            def _():
                z_rdma(k, my_z - t, 1, ze_send.at[k, t], ze_recv.at[k, t]).start()

            @pl.when((my_z > 0) & (my_z + t <= N_Z - 1))
            def _():
                z_rdma(k, my_z + t, -1, zw_send.at[k, t], zw_recv.at[k, t]).start()

        def z_waits(k, t):
            @pl.when(my_z >= t + 1)
            def _():
                z_rdma(k, my_z - 1 - t, -1, ze_send.at[k, t], ze_recv.at[k, t]).wait_recv()
                x_push(k, my_z - 1 - t).start()

            @pl.when(my_z <= N_Z - 2 - t)
            def _():
                z_rdma(k, my_z + 1 + t, 1, zw_send.at[k, t], zw_recv.at[k, t]).wait_recv()
                x_push(k, my_z + 1 + t).start()

            @pl.when((my_z < N_Z - 1) & (my_z >= t))
            def _():
                z_rdma(k, my_z - t, 1, ze_send.at[k, t], ze_recv.at[k, t]).wait_send()

            @pl.when((my_z > 0) & (my_z + t <= N_Z - 1))
            def _():
                z_rdma(k, my_z + t, -1, zw_send.at[k, t], zw_recv.at[k, t]).wait_send()

        for s in range(N_Y - 1):
            p1_start(0, s)
            p1_wait(0, s)
        p1_finish(0)

        for k in range(K):
            nxt = k + 1
            for t in range(N_Z - 1):
                z_sends(k, t)
                if nxt < K:
                    if t > 0:
                        p1_wait(nxt, t - 1)
                    p1_start(nxt, t)
                z_waits(k, t)
            if nxt < K:
                p1_wait(nxt, N_Y - 2)
                p1_finish(nxt)

        for k in range(K):
            for o in range(N_Z):
                x_recv_desc(k, o).wait_recv()
                x_push(k, o).wait_send()

    return pl.pallas_call(
        body,
        out_shape=jax.ShapeDtypeStruct((m, chunk), jnp.float32),
        in_specs=[pl.BlockSpec(memory_space=pl.ANY)],
        out_specs=pl.BlockSpec(memory_space=pltpu.MemorySpace.VMEM),
        scratch_shapes=[
            pltpu.VMEM((K, N_Y - 1, sub, chunk), jnp.float32),
            pltpu.VMEM((K, sub, chunk), jnp.float32),
            pltpu.SemaphoreType.DMA((K,)),
            pltpu.SemaphoreType.DMA((K, N_Y - 1)),
            pltpu.SemaphoreType.DMA((K, N_Y - 1)),
            pltpu.SemaphoreType.DMA((K, N_Z - 1)),
            pltpu.SemaphoreType.DMA((K, N_Z - 1)),
            pltpu.SemaphoreType.DMA((K, N_Z - 1)),
            pltpu.SemaphoreType.DMA((K, N_Z - 1)),
            pltpu.SemaphoreType.DMA((K, N_Z)),
            pltpu.SemaphoreType.DMA((K, N_Z)),
        ],
        compiler_params=pltpu.CompilerParams(
            collective_id=0,
            vmem_limit_bytes=63 * 1024 * 1024,
        ),
    )(x)
